# baseline (speedup 1.0000x reference)
import functools

import jax
import jax.numpy as jnp
from jax import lax
from jax.experimental import pallas as pl
from jax.experimental.pallas import tpu as pltpu
from jax.experimental.pallas import tpu_sc as plsc

B = 4096
N = 16384
L = 16
NC = 2
NS = 16
NW = NC * NS
ROWS_PER_W = B // NW


@functools.cache
def _build(nnzp: int):
    mesh = plsc.VectorSubcoreMesh(
        core_axis_name="c", subcore_axis_name="s", num_cores=NC, num_subcores=NS
    )

    @functools.partial(
        pl.kernel,
        out_type=jax.ShapeDtypeStruct((B, N), jnp.float32),
        mesh=mesh,
        compiler_params=pltpu.CompilerParams(needs_layout_passes=False),
        scratch_types=[
            pltpu.VMEM((nnzp,), jnp.int32),
            pltpu.VMEM((nnzp,), jnp.int32),
            pltpu.VMEM((nnzp,), jnp.float32),
            pltpu.VMEM((N,), jnp.float32),
            pltpu.VMEM((N,), jnp.float32),
            pltpu.VMEM((N,), jnp.float32),
        ],
    )
    def sc_kernel(inputs_hbm, w3_hbm, b_hbm, vel_hbm, i_hbm, j_hbm, out_hbm,
                  jv, iv, vals, xv, yv, biasv):
        wid = lax.axis_index("s") * NC + lax.axis_index("c")

        pltpu.sync_copy(j_hbm, jv)
        pltpu.sync_copy(i_hbm, iv)
        pltpu.sync_copy(w3_hbm, yv.at[pl.ds(0, nnzp)])
        pltpu.sync_copy(vel_hbm, xv)
        pltpu.sync_copy(b_hbm, biasv)

        def val_body(t, carry):
            o = t * L
            g = plsc.load_gather(xv, [jv[pl.ds(o, L)]])
            vals[pl.ds(o, L)] = yv[pl.ds(o, L)] * g
            return carry

        lax.fori_loop(0, nnzp // L, val_body, 0)

        def row_body(r, carry):
            row = wid * ROWS_PER_W + r
            pltpu.sync_copy(inputs_hbm.at[row], xv)

            def bias_body(i, c):
                o = i * L
                yv[pl.ds(o, L)] = biasv[pl.ds(o, L)]
                return c

            lax.fori_loop(0, N // L, bias_body, 0)

            def k_body(t, c):
                o = t * L
                g = plsc.load_gather(xv, [jv[pl.ds(o, L)]])
                plsc.addupdate_scatter(yv, [iv[pl.ds(o, L)]], vals[pl.ds(o, L)] * g)
                return c

            lax.fori_loop(0, nnzp // L, k_body, 0)
            pltpu.sync_copy(yv, out_hbm.at[row])
            return carry

        lax.fori_loop(0, ROWS_PER_W, row_body, 0)

    return sc_kernel


def kernel(inputs, W3, b, velocity, I, J):
    nnz = W3.shape[0]
    nnzp = ((nnz + L - 1) // L) * L
    pad = nnzp - nnz
    Jp = jnp.concatenate([J.astype(jnp.int32), jnp.zeros((pad,), jnp.int32)])
    Ip = jnp.concatenate([I.astype(jnp.int32), jnp.zeros((pad,), jnp.int32)])
    W3p = jnp.concatenate([W3, jnp.zeros((pad,), jnp.float32)])
    return _build(nnzp)(inputs, W3p, b, velocity, Ip, Jp)


# speedup vs baseline: 1.4208x; 1.4208x over previous
import functools

import jax
import jax.numpy as jnp
from jax import lax
from jax.experimental import pallas as pl
from jax.experimental.pallas import tpu as pltpu
from jax.experimental.pallas import tpu_sc as plsc

B = 4096
N = 16384
L = 16
NC = 2
NS = 16
NW = NC * NS
ROWS_PER_W = B // NW
KU = 8
BU = 8
JBITS = 14
JMASK = (1 << JBITS) - 1


@functools.cache
def _build(nnzp: int):
    mesh = plsc.VectorSubcoreMesh(
        core_axis_name="c", subcore_axis_name="s", num_cores=NC, num_subcores=NS
    )

    @functools.partial(
        pl.kernel,
        out_type=jax.ShapeDtypeStruct((B, N), jnp.float32),
        mesh=mesh,
        compiler_params=pltpu.CompilerParams(needs_layout_passes=False),
        scratch_types=[
            pltpu.VMEM((nnzp,), jnp.int32),
            pltpu.VMEM((nnzp,), jnp.float32),
            pltpu.VMEM((N,), jnp.float32),
            pltpu.VMEM((N,), jnp.float32),
            pltpu.VMEM((N,), jnp.float32),
            pltpu.VMEM((N,), jnp.float32),
            pltpu.VMEM((N,), jnp.float32),
            pltpu.SemaphoreType.DMA,
            pltpu.SemaphoreType.DMA,
            pltpu.SemaphoreType.DMA,
            pltpu.SemaphoreType.DMA,
        ],
    )
    def sc_kernel(inputs_hbm, w3_hbm, b_hbm, vel_hbm, packed_hbm, out_hbm,
                  packed, vals, biasv, x0, x1, y0, y1,
                  sx0, sx1, sy0, sy1):
        wid = lax.axis_index("s") * NC + lax.axis_index("c")
        row0 = wid * ROWS_PER_W

        pltpu.sync_copy(packed_hbm, packed)
        pltpu.sync_copy(w3_hbm, y0.at[pl.ds(0, nnzp)])
        pltpu.sync_copy(vel_hbm, x0)
        pltpu.sync_copy(b_hbm, biasv)

        def val_body(t, c):
            base = t * (L * KU)
            for u in range(KU):
                o = base + u * L
                pk = packed[pl.ds(o, L)]
                jv = lax.bitwise_and(pk, JMASK)
                g = plsc.load_gather(x0, [jv])
                vals[pl.ds(o, L)] = y0[pl.ds(o, L)] * g
            return c

        lax.fori_loop(0, nnzp // (L * KU), val_body, 0)

        def bias_init(ybuf):
            def bias_body(i, c):
                base = i * (L * BU)
                for u in range(BU):
                    o = base + u * L
                    ybuf[pl.ds(o, L)] = biasv[pl.ds(o, L)]
                return c

            lax.fori_loop(0, N // (L * BU), bias_body, 0)

        def k_loop(xbuf, ybuf):
            def k_body(t, c):
                base = t * (L * KU)
                for u in range(KU):
                    o = base + u * L
                    pk = packed[pl.ds(o, L)]
                    jv = lax.bitwise_and(pk, JMASK)
                    iv = lax.shift_right_logical(pk, JBITS)
                    g = plsc.load_gather(xbuf, [jv])
                    plsc.addupdate_scatter(ybuf, [iv], vals[pl.ds(o, L)] * g)
                return c

            lax.fori_loop(0, nnzp // (L * KU), k_body, 0)

        pltpu.async_copy(inputs_hbm.at[row0], x0, sx0)

        def pair_body(p, c):
            ra = row0 + 2 * p
            rb = ra + 1
            pltpu.make_async_copy(inputs_hbm.at[ra], x0, sx0).wait()
            pltpu.async_copy(inputs_hbm.at[rb], x1, sx1)

            @pl.when(p > 0)
            def _():
                pltpu.make_async_copy(y0, out_hbm.at[ra - 2], sy0).wait()

            bias_init(y0)
            k_loop(x0, y0)
            pltpu.async_copy(y0, out_hbm.at[ra], sy0)

            pltpu.make_async_copy(inputs_hbm.at[rb], x1, sx1).wait()

            @pl.when(p < ROWS_PER_W // 2 - 1)
            def _():
                pltpu.async_copy(inputs_hbm.at[ra + 2], x0, sx0)

            @pl.when(p > 0)
            def _():
                pltpu.make_async_copy(y1, out_hbm.at[rb - 2], sy1).wait()

            bias_init(y1)
            k_loop(x1, y1)
            pltpu.async_copy(y1, out_hbm.at[rb], sy1)
            return c

        lax.fori_loop(0, ROWS_PER_W // 2, pair_body, 0)
        last = row0 + ROWS_PER_W
        pltpu.make_async_copy(y0, out_hbm.at[last - 2], sy0).wait()
        pltpu.make_async_copy(y1, out_hbm.at[last - 1], sy1).wait()

    return sc_kernel


def kernel(inputs, W3, b, velocity, I, J):
    nnz = W3.shape[0]
    chunk = L * KU
    nnzp = ((nnz + chunk - 1) // chunk) * chunk
    pad = nnzp - nnz
    packed = jnp.left_shift(I.astype(jnp.int32), JBITS) | J.astype(jnp.int32)
    packed = jnp.concatenate([packed, jnp.zeros((pad,), jnp.int32)])
    W3p = jnp.concatenate([W3, jnp.zeros((pad,), jnp.float32)])
    return _build(nnzp)(inputs, W3p, b, velocity, packed)


# speedup vs baseline: 4.1534x; 2.9233x over previous
import functools

import jax
import jax.numpy as jnp
from jax import lax
from jax.experimental import pallas as pl
from jax.experimental.pallas import tpu as pltpu
from jax.experimental.pallas import tpu_sc as plsc

B = 4096
N = 16384
L = 16
NC = 2
NS = 16
NW = NC * NS
ROWS_PER_W = B // NW
KU = 8
BU = 8
JBITS = 14
JMASK = (1 << JBITS) - 1


@functools.cache
def _build(nnzp: int):
    mesh = plsc.VectorSubcoreMesh(
        core_axis_name="c", subcore_axis_name="s", num_cores=NC, num_subcores=NS
    )

    @functools.partial(
        pl.kernel,
        out_type=jax.ShapeDtypeStruct((B, N), jnp.float32),
        mesh=mesh,
        compiler_params=pltpu.CompilerParams(needs_layout_passes=False),
        scratch_types=[
            pltpu.VMEM((nnzp,), jnp.int32),
            pltpu.VMEM((nnzp,), jnp.float32),
            pltpu.VMEM((N,), jnp.float32),
            pltpu.VMEM((N,), jnp.float32),
            pltpu.VMEM((N,), jnp.float32),
            pltpu.VMEM((N,), jnp.float32),
            pltpu.VMEM((N,), jnp.float32),
            pltpu.SemaphoreType.DMA,
            pltpu.SemaphoreType.DMA,
            pltpu.SemaphoreType.DMA,
            pltpu.SemaphoreType.DMA,
        ],
    )
    def sc_kernel(inputs_hbm, w3_hbm, b_hbm, vel_hbm, packed_hbm, out_hbm,
                  packed, vals, biasv, x0, x1, y0, y1,
                  sx0, sx1, sy0, sy1):
        wid = lax.axis_index("s") * NC + lax.axis_index("c")
        row0 = wid * ROWS_PER_W

        pltpu.sync_copy(packed_hbm, packed)
        pltpu.sync_copy(w3_hbm, y0.at[pl.ds(0, nnzp)])
        pltpu.sync_copy(vel_hbm, x0)
        pltpu.sync_copy(b_hbm, biasv)

        @plsc.parallel_loop(0, nnzp // L, unroll=KU)
        def val_body(t):
            o = t * L
            pk = packed[pl.ds(o, L)]
            jv = lax.bitwise_and(pk, JMASK)
            g = plsc.load_gather(x0, [jv])
            vals[pl.ds(o, L)] = y0[pl.ds(o, L)] * g

        def bias_init(ybuf):
            @plsc.parallel_loop(0, N // L, unroll=BU)
            def bias_body(i):
                o = i * L
                ybuf[pl.ds(o, L)] = biasv[pl.ds(o, L)]

        def k_loop(xbuf, ybuf):
            @plsc.parallel_loop(0, nnzp // L, unroll=KU)
            def k_body(t):
                o = t * L
                pk = packed[pl.ds(o, L)]
                jv = lax.bitwise_and(pk, JMASK)
                iv = lax.shift_right_logical(pk, JBITS)
                g = plsc.load_gather(xbuf, [jv])
                plsc.addupdate_scatter(ybuf, [iv], vals[pl.ds(o, L)] * g)

        pltpu.async_copy(inputs_hbm.at[row0], x0, sx0)

        def pair_body(p, c):
            ra = row0 + 2 * p
            rb = ra + 1
            pltpu.make_async_copy(inputs_hbm.at[ra], x0, sx0).wait()
            pltpu.async_copy(inputs_hbm.at[rb], x1, sx1)

            @pl.when(p > 0)
            def _():
                pltpu.make_async_copy(y0, out_hbm.at[ra - 2], sy0).wait()

            bias_init(y0)
            k_loop(x0, y0)
            pltpu.async_copy(y0, out_hbm.at[ra], sy0)

            pltpu.make_async_copy(inputs_hbm.at[rb], x1, sx1).wait()

            @pl.when(p < ROWS_PER_W // 2 - 1)
            def _():
                pltpu.async_copy(inputs_hbm.at[ra + 2], x0, sx0)

            @pl.when(p > 0)
            def _():
                pltpu.make_async_copy(y1, out_hbm.at[rb - 2], sy1).wait()

            bias_init(y1)
            k_loop(x1, y1)
            pltpu.async_copy(y1, out_hbm.at[rb], sy1)
            return c

        lax.fori_loop(0, ROWS_PER_W // 2, pair_body, 0)
        last = row0 + ROWS_PER_W
        pltpu.make_async_copy(y0, out_hbm.at[last - 2], sy0).wait()
        pltpu.make_async_copy(y1, out_hbm.at[last - 1], sy1).wait()

    return sc_kernel


def kernel(inputs, W3, b, velocity, I, J):
    nnz = W3.shape[0]
    chunk = L * KU
    nnzp = ((nnz + chunk - 1) // chunk) * chunk
    pad = nnzp - nnz
    packed = jnp.left_shift(I.astype(jnp.int32), JBITS) | J.astype(jnp.int32)
    packed = jnp.concatenate([packed, jnp.zeros((pad,), jnp.int32)])
    W3p = jnp.concatenate([W3, jnp.zeros((pad,), jnp.float32)])
    return _build(nnzp)(inputs, W3p, b, velocity, packed)
